# baseline (speedup 1.0000x reference)
import jax
import jax.numpy as jnp
from jax import lax
from jax.experimental import pallas as pl
from jax.experimental.pallas import tpu as pltpu
from jax.experimental.pallas import tpu_sc as plsc

N_ROWS = 1_000_000
DIM = 32
BATCH = 16384

SEG = 262144
SEG_SHIFT = 18
SEG_MASK = SEG - 1
SUP = 128

_NC = 2
_NS = 16
_NW = _NC * _NS
_CHUNK = 128
_B_PER_W = BATCH // _NW
_N_CHUNKS = _B_PER_W // _CHUNK
_L = 16


_S_PER_W = SEG // _NW
_NBLK = _S_PER_W // _CHUNK
_SEG3 = N_ROWS - 3 * SEG
_SEG3_FULL = (_SEG3 // _CHUNK) * _CHUNK


def _spack_body(zt_hbm, t2_hbm, tin0, tin1, tout0, tout1,
                sem_i0, sem_i1, sem_o0, sem_o1):
    wid = lax.axis_index("s") * _NC + lax.axis_index("c")
    base = wid * _S_PER_W
    iota = lax.iota(jnp.int32, _L)
    tins = (tin0, tin1)
    touts = (tout0, tout1)
    sems_i = (sem_i0, sem_i1)
    sems_o = (sem_o0, sem_o1)
    rows16 = [iota + g * _L for g in range(_CHUNK // _L)]

    def start_in(s0, buf):
        for q in range(3):
            pltpu.async_copy(
                zt_hbm.at[:, pl.ds(q * SEG + s0, _CHUNK)],
                tins[buf].at[q], sems_i[buf])

        @pl.when(s0 < _SEG3_FULL)
        def _():
            pltpu.async_copy(
                zt_hbm.at[:, pl.ds(3 * SEG + s0, _CHUNK)],
                tins[buf].at[3], sems_i[buf])

    def wait_in(s0, buf):
        for q in range(3):
            pltpu.make_async_copy(
                zt_hbm.at[:, pl.ds(q * SEG + s0, _CHUNK)],
                tins[buf].at[q], sems_i[buf]).wait()

        @pl.when(s0 < _SEG3_FULL)
        def _():
            pltpu.make_async_copy(
                zt_hbm.at[:, pl.ds(3 * SEG + s0, _CHUNK)],
                tins[buf].at[3], sems_i[buf]).wait()

    def drain_out(s0, buf):
        pltpu.make_async_copy(
            touts[buf], t2_hbm.at[pl.ds(s0, _CHUNK)], sems_o[buf]).wait()

    def fold(buf):
        tin = tins[buf]
        tout = touts[buf]
        for q in range(4):
            for c in range(DIM):
                colv = jnp.full((_L,), q * DIM + c, jnp.int32)
                for g in range(_CHUNK // _L):
                    vals = tin[q, c, pl.ds(g * _L, _L)]
                    plsc.store_scatter(tout, [rows16[g], colv], vals)

    start_in(base, 0)
    start_in(base + _CHUNK, 1)

    @pl.loop(0, _NBLK, step=2)
    def _blocks(i):
        for sub in range(2):
            b = i + sub
            s0 = base + b * _CHUNK
            wait_in(s0, sub)

            @pl.when(b >= 2)
            def _():
                drain_out(s0 - 2 * _CHUNK, sub)

            fold(sub)
            pltpu.async_copy(
                touts[sub], t2_hbm.at[pl.ds(s0, _CHUNK)], sems_o[sub])

            @pl.when(b + 2 < _NBLK)
            def _():
                start_in(s0 + 2 * _CHUNK, sub)

    drain_out(base + (_NBLK - 2) * _CHUNK, 0)
    drain_out(base + (_NBLK - 1) * _CHUNK, 1)


def _pack(zt):
    mesh = plsc.VectorSubcoreMesh(core_axis_name="c", subcore_axis_name="s")
    run = pl.kernel(
        _spack_body,
        out_type=jax.ShapeDtypeStruct((SEG, SUP), jnp.float32),
        mesh=mesh,
        scratch_types=[
            pltpu.VMEM((4, DIM, _CHUNK), jnp.float32),
            pltpu.VMEM((4, DIM, _CHUNK), jnp.float32),
            pltpu.VMEM((_CHUNK, SUP), jnp.float32),
            pltpu.VMEM((_CHUNK, SUP), jnp.float32),
            pltpu.SemaphoreType.DMA,
            pltpu.SemaphoreType.DMA,
            pltpu.SemaphoreType.DMA,
            pltpu.SemaphoreType.DMA,
        ],
        compiler_params=pltpu.CompilerParams(needs_layout_passes=False, disable_bounds_checks=True, disable_semaphore_checks=True),
    )
    return run(zt)


def _gather_body(idx_hbm, table_hbm, out_hbm, idx_v, sup_v, rem_v,
                 big0, big1, out_v, sem0, sem1):
    wid = lax.axis_index("s") * _NC + lax.axis_index("c")
    pltpu.sync_copy(idx_hbm.at[pl.ds(wid * _N_CHUNKS, _N_CHUNKS)], idx_v)

    for t in range(_N_CHUNKS):
        for k in range(_CHUNK // _L):
            v = idx_v[t, pl.ds(k * _L, _L)]
            sup_v[t, pl.ds(k * _L, _L)] = v & SEG_MASK
            rem_v[pl.ds((t * (_CHUNK // _L) + k) * _L, _L)] = v >> SEG_SHIFT

    bufs = (big0, big1)
    sems = (sem0, sem1)
    iota = lax.iota(jnp.int32, _L)

    def select_chunk(t, buf):
        def group_body(g, carry):
            lrow = g * _L + iota
            orow = t * _CHUNK + lrow
            rem16 = rem_v[pl.ds(t * _CHUNK + g * _L, _L)]
            col_base = rem16 * DIM
            for c in range(DIM):
                vals = plsc.load_gather(buf, [lrow, col_base + c])
                plsc.store_scatter(
                    out_v, [orow, jnp.full((_L,), c, jnp.int32)], vals)
            return carry

        lax.fori_loop(0, _CHUNK // _L, group_body, 0)

    def start(t):
        return pltpu.async_copy(
            table_hbm.at[sup_v.at[t]], bufs[t % 2], sems[t % 2])

    copies = {0: start(0)}
    for t in range(_N_CHUNKS):
        if t + 1 < _N_CHUNKS:
            copies[t + 1] = start(t + 1)
        copies[t].wait()
        select_chunk(t, bufs[t % 2])

    pltpu.sync_copy(out_v, out_hbm.at[pl.ds(wid * _B_PER_W, _B_PER_W)])


@jax.jit
def kernel(ixs, z):
    idx2d = ixs.astype(jnp.int32).reshape(BATCH // _CHUNK, _CHUNK)
    table2 = _pack(z.T)
    tail = jax.lax.slice(z, (N_ROWS - (_SEG3 - _SEG3_FULL), 0), (N_ROWS, DIM))
    table2 = jax.lax.dynamic_update_slice(table2, tail, (_SEG3_FULL, 96))
    mesh = plsc.VectorSubcoreMesh(core_axis_name="c", subcore_axis_name="s")
    run = pl.kernel(
        _gather_body,
        out_type=jax.ShapeDtypeStruct((BATCH, DIM), jnp.float32),
        mesh=mesh,
        scratch_types=[
            pltpu.VMEM((_N_CHUNKS, _CHUNK), jnp.int32),
            pltpu.VMEM((_N_CHUNKS, _CHUNK), jnp.int32),
            pltpu.VMEM((_B_PER_W,), jnp.int32),
            pltpu.VMEM((_CHUNK, SUP), jnp.float32),
            pltpu.VMEM((_CHUNK, SUP), jnp.float32),
            pltpu.VMEM((_B_PER_W, DIM), jnp.float32),
            pltpu.SemaphoreType.DMA,
            pltpu.SemaphoreType.DMA,
        ],
        compiler_params=pltpu.CompilerParams(needs_layout_passes=False, disable_bounds_checks=True, disable_semaphore_checks=True),
    )
    return run(idx2d, table2)


# speedup vs baseline: 2.0029x; 2.0029x over previous
import jax
import jax.numpy as jnp
from jax import lax
from jax.experimental import pallas as pl
from jax.experimental.pallas import tpu as pltpu
from jax.experimental.pallas import tpu_sc as plsc

N_ROWS = 1_000_000
DIM = 32
BATCH = 16384

SEG = 262144
SEG_SHIFT = 18
SEG_MASK = SEG - 1
SUP = 128

_NC = 2
_NS = 16
_NW = _NC * _NS
_CHUNK = 128
_B_PER_W = BATCH // _NW
_N_CHUNKS = _B_PER_W // _CHUNK
_L = 16


_TBLK = 1024
_TGRID = SEG // _TBLK
_ZCB = N_ROWS // _TBLK
_TAIL_I = (N_ROWS - 3 * SEG) // _TBLK


def _pack_body(in0, in1, in2, in3, o_ref):
    i = pl.program_id(0)
    o_ref[:, 0:32] = in0[...].T
    o_ref[:, 32:64] = in1[...].T
    o_ref[:, 64:96] = in2[...].T

    @pl.when(i < _TAIL_I)
    def _():
        o_ref[:, 96:128] = in3[...].T

    @pl.when(i == _TAIL_I)
    def _():
        o_ref[0:576, 96:128] = in3[:, 0:576].T


def _pack(zt):
    return pl.pallas_call(
        _pack_body,
        grid=(_TGRID,),
        in_specs=[
            pl.BlockSpec((DIM, _TBLK), lambda i: (0, i)),
            pl.BlockSpec((DIM, _TBLK), lambda i: (0, i + _TGRID)),
            pl.BlockSpec((DIM, _TBLK), lambda i: (0, i + 2 * _TGRID)),
            pl.BlockSpec((DIM, _TBLK),
                         lambda i: (0, jnp.minimum(i + 3 * _TGRID, _ZCB))),
        ],
        out_specs=pl.BlockSpec((_TBLK, SUP), lambda i: (i, 0)),
        out_shape=jax.ShapeDtypeStruct((SEG, SUP), jnp.float32),
    )(zt, zt, zt, zt)


def _gather_body(idx_hbm, table_hbm, out_hbm, idx_v, sup_v, rem_v,
                 big0, big1, out_v, sem0, sem1):
    wid = lax.axis_index("s") * _NC + lax.axis_index("c")
    pltpu.sync_copy(idx_hbm.at[pl.ds(wid * _N_CHUNKS, _N_CHUNKS)], idx_v)

    for t in range(_N_CHUNKS):
        for k in range(_CHUNK // _L):
            v = idx_v[t, pl.ds(k * _L, _L)]
            sup_v[t, pl.ds(k * _L, _L)] = v & SEG_MASK
            rem_v[pl.ds((t * (_CHUNK // _L) + k) * _L, _L)] = v >> SEG_SHIFT

    bufs = (big0, big1)
    sems = (sem0, sem1)
    iota = lax.iota(jnp.int32, _L)

    def select_chunk(t, buf):
        def group_body(g, carry):
            lrow = g * _L + iota
            orow = t * _CHUNK + lrow
            rem16 = rem_v[pl.ds(t * _CHUNK + g * _L, _L)]
            col_base = rem16 * DIM
            for c in range(DIM):
                vals = plsc.load_gather(buf, [lrow, col_base + c])
                plsc.store_scatter(
                    out_v, [orow, jnp.full((_L,), c, jnp.int32)], vals)
            return carry

        lax.fori_loop(0, _CHUNK // _L, group_body, 0)

    def start(t):
        return pltpu.async_copy(
            table_hbm.at[sup_v.at[t]], bufs[t % 2], sems[t % 2])

    copies = {0: start(0)}
    for t in range(_N_CHUNKS):
        if t + 1 < _N_CHUNKS:
            copies[t + 1] = start(t + 1)
        copies[t].wait()
        select_chunk(t, bufs[t % 2])

    pltpu.sync_copy(out_v, out_hbm.at[pl.ds(wid * _B_PER_W, _B_PER_W)])


@jax.jit
def kernel(ixs, z):
    idx2d = ixs.astype(jnp.int32).reshape(BATCH // _CHUNK, _CHUNK)
    table2 = _pack(z.T)
    mesh = plsc.VectorSubcoreMesh(core_axis_name="c", subcore_axis_name="s")
    run = pl.kernel(
        _gather_body,
        out_type=jax.ShapeDtypeStruct((BATCH, DIM), jnp.float32),
        mesh=mesh,
        scratch_types=[
            pltpu.VMEM((_N_CHUNKS, _CHUNK), jnp.int32),
            pltpu.VMEM((_N_CHUNKS, _CHUNK), jnp.int32),
            pltpu.VMEM((_B_PER_W,), jnp.int32),
            pltpu.VMEM((_CHUNK, SUP), jnp.float32),
            pltpu.VMEM((_CHUNK, SUP), jnp.float32),
            pltpu.VMEM((_B_PER_W, DIM), jnp.float32),
            pltpu.SemaphoreType.DMA,
            pltpu.SemaphoreType.DMA,
        ],
        compiler_params=pltpu.CompilerParams(needs_layout_passes=False, disable_bounds_checks=True, disable_semaphore_checks=True),
    )
    return run(idx2d, table2)


# speedup vs baseline: 2.3064x; 1.1515x over previous
import jax
import jax.numpy as jnp
from jax import lax
from jax.experimental import pallas as pl
from jax.experimental.pallas import tpu as pltpu
from jax.experimental.pallas import tpu_sc as plsc

N_ROWS = 1_000_000
DIM = 32
BATCH = 16384

SEG = 262144
SEG_SHIFT = 18
SEG_MASK = SEG - 1
SUP = 128

_NC = 2
_NS = 16
_NW = _NC * _NS
_CHUNK = 128
_B_PER_W = BATCH // _NW
_N_CHUNKS = _B_PER_W // _CHUNK
_L = 16


_TBLK = 2048
_TGRID = SEG // _TBLK
_ZCB = N_ROWS // _TBLK
_TAIL_I = (N_ROWS - 3 * SEG) // _TBLK


def _pack_body(in0, in1, in2, in3, o_ref):
    i = pl.program_id(0)
    o_ref[:, 0:32] = in0[...].T
    o_ref[:, 32:64] = in1[...].T
    o_ref[:, 64:96] = in2[...].T

    @pl.when(i < _TAIL_I)
    def _():
        o_ref[:, 96:128] = in3[...].T

    @pl.when(i == _TAIL_I)
    def _():
        o_ref[0:576, 96:128] = in3[:, 0:576].T


def _pack(zt):
    return pl.pallas_call(
        _pack_body,
        grid=(_TGRID,),
        in_specs=[
            pl.BlockSpec((DIM, _TBLK), lambda i: (0, i)),
            pl.BlockSpec((DIM, _TBLK), lambda i: (0, i + _TGRID)),
            pl.BlockSpec((DIM, _TBLK), lambda i: (0, i + 2 * _TGRID)),
            pl.BlockSpec((DIM, _TBLK),
                         lambda i: (0, jnp.minimum(i + 3 * _TGRID, _ZCB))),
        ],
        out_specs=pl.BlockSpec((_TBLK, SUP), lambda i: (i, 0)),
        out_shape=jax.ShapeDtypeStruct((SEG, SUP), jnp.float32),
    )(zt, zt, zt, zt)


def _gather_body(idx_hbm, table_hbm, out_hbm, idx_v, sup_v, rem_v,
                 big0, big1, out_v, sem0, sem1):
    wid = lax.axis_index("s") * _NC + lax.axis_index("c")
    pltpu.sync_copy(idx_hbm.at[pl.ds(wid * _N_CHUNKS, _N_CHUNKS)], idx_v)

    for t in range(_N_CHUNKS):
        for k in range(_CHUNK // _L):
            v = idx_v[t, pl.ds(k * _L, _L)]
            sup_v[t, pl.ds(k * _L, _L)] = v & SEG_MASK
            rem_v[pl.ds((t * (_CHUNK // _L) + k) * _L, _L)] = v >> SEG_SHIFT

    bufs = (big0, big1)
    sems = (sem0, sem1)
    iota = lax.iota(jnp.int32, _L)

    def select_chunk(t, buf):
        def group_body(g, carry):
            lrow = g * _L + iota
            orow = t * _CHUNK + lrow
            rem16 = rem_v[pl.ds(t * _CHUNK + g * _L, _L)]
            col_base = rem16 * DIM
            for c in range(DIM):
                vals = plsc.load_gather(buf, [lrow, col_base + c])
                plsc.store_scatter(
                    out_v, [orow, jnp.full((_L,), c, jnp.int32)], vals)
            return carry

        lax.fori_loop(0, _CHUNK // _L, group_body, 0)

    def start(t):
        return pltpu.async_copy(
            table_hbm.at[sup_v.at[t]], bufs[t % 2], sems[t % 2])

    copies = {0: start(0)}
    for t in range(_N_CHUNKS):
        if t + 1 < _N_CHUNKS:
            copies[t + 1] = start(t + 1)
        copies[t].wait()
        select_chunk(t, bufs[t % 2])

    pltpu.sync_copy(out_v, out_hbm.at[pl.ds(wid * _B_PER_W, _B_PER_W)])


@jax.jit
def kernel(ixs, z):
    idx2d = ixs.astype(jnp.int32).reshape(BATCH // _CHUNK, _CHUNK)
    table2 = _pack(z.T)
    mesh = plsc.VectorSubcoreMesh(core_axis_name="c", subcore_axis_name="s")
    run = pl.kernel(
        _gather_body,
        out_type=jax.ShapeDtypeStruct((BATCH, DIM), jnp.float32),
        mesh=mesh,
        scratch_types=[
            pltpu.VMEM((_N_CHUNKS, _CHUNK), jnp.int32),
            pltpu.VMEM((_N_CHUNKS, _CHUNK), jnp.int32),
            pltpu.VMEM((_B_PER_W,), jnp.int32),
            pltpu.VMEM((_CHUNK, SUP), jnp.float32),
            pltpu.VMEM((_CHUNK, SUP), jnp.float32),
            pltpu.VMEM((_B_PER_W, DIM), jnp.float32),
            pltpu.SemaphoreType.DMA,
            pltpu.SemaphoreType.DMA,
        ],
        compiler_params=pltpu.CompilerParams(needs_layout_passes=False, disable_bounds_checks=True, disable_semaphore_checks=True),
    )
    return run(idx2d, table2)


# speedup vs baseline: 2.4290x; 1.0531x over previous
import jax
import jax.numpy as jnp
from jax import lax
from jax.experimental import pallas as pl
from jax.experimental.pallas import tpu as pltpu
from jax.experimental.pallas import tpu_sc as plsc

N_ROWS = 1_000_000
DIM = 32
BATCH = 16384

SEG = 262144
SEG_SHIFT = 18
SEG_MASK = SEG - 1
SUP = 128

_NC = 2
_NS = 16
_NW = _NC * _NS
_CHUNK = 128
_B_PER_W = BATCH // _NW
_N_CHUNKS = _B_PER_W // _CHUNK
_L = 16


_TBLK = 4096
_TGRID = SEG // _TBLK
_ZCB = N_ROWS // _TBLK
_TAIL_I = (N_ROWS - 3 * SEG) // _TBLK


def _pack_body(in0, in1, in2, in3, o_ref):
    i = pl.program_id(0)
    o_ref[:, 0:32] = in0[...].T
    o_ref[:, 32:64] = in1[...].T
    o_ref[:, 64:96] = in2[...].T

    @pl.when(i < _TAIL_I)
    def _():
        o_ref[:, 96:128] = in3[...].T

    @pl.when(i == _TAIL_I)
    def _():
        o_ref[0:576, 96:128] = in3[:, 0:576].T


def _pack(zt):
    return pl.pallas_call(
        _pack_body,
        grid=(_TGRID,),
        in_specs=[
            pl.BlockSpec((DIM, _TBLK), lambda i: (0, i)),
            pl.BlockSpec((DIM, _TBLK), lambda i: (0, i + _TGRID)),
            pl.BlockSpec((DIM, _TBLK), lambda i: (0, i + 2 * _TGRID)),
            pl.BlockSpec((DIM, _TBLK),
                         lambda i: (0, jnp.minimum(i + 3 * _TGRID, _ZCB))),
        ],
        out_specs=pl.BlockSpec((_TBLK, SUP), lambda i: (i, 0)),
        out_shape=jax.ShapeDtypeStruct((SEG, SUP), jnp.float32),
    )(zt, zt, zt, zt)


def _gather_body(idx_hbm, table_hbm, out_hbm, idx_v, sup_v, rem_v,
                 big0, big1, out_v, sem0, sem1):
    wid = lax.axis_index("s") * _NC + lax.axis_index("c")
    pltpu.sync_copy(idx_hbm.at[pl.ds(wid * _N_CHUNKS, _N_CHUNKS)], idx_v)

    for t in range(_N_CHUNKS):
        for k in range(_CHUNK // _L):
            v = idx_v[t, pl.ds(k * _L, _L)]
            sup_v[t, pl.ds(k * _L, _L)] = v & SEG_MASK
            rem_v[pl.ds((t * (_CHUNK // _L) + k) * _L, _L)] = v >> SEG_SHIFT

    bufs = (big0, big1)
    sems = (sem0, sem1)
    iota = lax.iota(jnp.int32, _L)

    def select_chunk(t, buf):
        def group_body(g, carry):
            lrow = g * _L + iota
            orow = t * _CHUNK + lrow
            rem16 = rem_v[pl.ds(t * _CHUNK + g * _L, _L)]
            col_base = rem16 * DIM
            for c in range(DIM):
                vals = plsc.load_gather(buf, [lrow, col_base + c])
                plsc.store_scatter(
                    out_v, [orow, jnp.full((_L,), c, jnp.int32)], vals)
            return carry

        lax.fori_loop(0, _CHUNK // _L, group_body, 0)

    def start(t):
        return pltpu.async_copy(
            table_hbm.at[sup_v.at[t]], bufs[t % 2], sems[t % 2])

    copies = {0: start(0)}
    for t in range(_N_CHUNKS):
        if t + 1 < _N_CHUNKS:
            copies[t + 1] = start(t + 1)
        copies[t].wait()
        select_chunk(t, bufs[t % 2])

    pltpu.sync_copy(out_v, out_hbm.at[pl.ds(wid * _B_PER_W, _B_PER_W)])


@jax.jit
def kernel(ixs, z):
    idx2d = ixs.astype(jnp.int32).reshape(BATCH // _CHUNK, _CHUNK)
    table2 = _pack(z.T)
    mesh = plsc.VectorSubcoreMesh(core_axis_name="c", subcore_axis_name="s")
    run = pl.kernel(
        _gather_body,
        out_type=jax.ShapeDtypeStruct((BATCH, DIM), jnp.float32),
        mesh=mesh,
        scratch_types=[
            pltpu.VMEM((_N_CHUNKS, _CHUNK), jnp.int32),
            pltpu.VMEM((_N_CHUNKS, _CHUNK), jnp.int32),
            pltpu.VMEM((_B_PER_W,), jnp.int32),
            pltpu.VMEM((_CHUNK, SUP), jnp.float32),
            pltpu.VMEM((_CHUNK, SUP), jnp.float32),
            pltpu.VMEM((_B_PER_W, DIM), jnp.float32),
            pltpu.SemaphoreType.DMA,
            pltpu.SemaphoreType.DMA,
        ],
        compiler_params=pltpu.CompilerParams(needs_layout_passes=False, disable_bounds_checks=True, disable_semaphore_checks=True),
    )
    return run(idx2d, table2)


# speedup vs baseline: 2.4816x; 1.0217x over previous
import jax
import jax.numpy as jnp
from jax import lax
from jax.experimental import pallas as pl
from jax.experimental.pallas import tpu as pltpu
from jax.experimental.pallas import tpu_sc as plsc

N_ROWS = 1_000_000
DIM = 32
BATCH = 16384

SEG = 262144
SEG_SHIFT = 18
SEG_MASK = SEG - 1
SUP = 128

_NC = 2
_NS = 16
_NW = _NC * _NS
_CHUNK = 128
_B_PER_W = BATCH // _NW
_N_CHUNKS = _B_PER_W // _CHUNK
_L = 16


_TBLK = 8192
_TGRID = SEG // _TBLK
_ZCB = N_ROWS // _TBLK
_TAIL_I = (N_ROWS - 3 * SEG) // _TBLK


def _pack_body(in0, in1, in2, in3, o_ref):
    i = pl.program_id(0)
    o_ref[:, 0:32] = in0[...].T
    o_ref[:, 32:64] = in1[...].T
    o_ref[:, 64:96] = in2[...].T

    @pl.when(i < _TAIL_I)
    def _():
        o_ref[:, 96:128] = in3[...].T

    @pl.when(i == _TAIL_I)
    def _():
        o_ref[0:576, 96:128] = in3[:, 0:576].T


def _pack(zt):
    return pl.pallas_call(
        _pack_body,
        grid=(_TGRID,),
        in_specs=[
            pl.BlockSpec((DIM, _TBLK), lambda i: (0, i)),
            pl.BlockSpec((DIM, _TBLK), lambda i: (0, i + _TGRID)),
            pl.BlockSpec((DIM, _TBLK), lambda i: (0, i + 2 * _TGRID)),
            pl.BlockSpec((DIM, _TBLK),
                         lambda i: (0, jnp.minimum(i + 3 * _TGRID, _ZCB))),
        ],
        out_specs=pl.BlockSpec((_TBLK, SUP), lambda i: (i, 0)),
        out_shape=jax.ShapeDtypeStruct((SEG, SUP), jnp.float32),
    )(zt, zt, zt, zt)


def _gather_body(idx_hbm, table_hbm, out_hbm, idx_v, sup_v, rem_v,
                 big0, big1, out_v, sem0, sem1):
    wid = lax.axis_index("s") * _NC + lax.axis_index("c")
    pltpu.sync_copy(idx_hbm.at[pl.ds(wid * _N_CHUNKS, _N_CHUNKS)], idx_v)

    for t in range(_N_CHUNKS):
        for k in range(_CHUNK // _L):
            v = idx_v[t, pl.ds(k * _L, _L)]
            sup_v[t, pl.ds(k * _L, _L)] = v & SEG_MASK
            rem_v[pl.ds((t * (_CHUNK // _L) + k) * _L, _L)] = v >> SEG_SHIFT

    bufs = (big0, big1)
    sems = (sem0, sem1)
    iota = lax.iota(jnp.int32, _L)

    def select_chunk(t, buf):
        def group_body(g, carry):
            lrow = g * _L + iota
            orow = t * _CHUNK + lrow
            rem16 = rem_v[pl.ds(t * _CHUNK + g * _L, _L)]
            col_base = rem16 * DIM
            for c in range(DIM):
                vals = plsc.load_gather(buf, [lrow, col_base + c])
                plsc.store_scatter(
                    out_v, [orow, jnp.full((_L,), c, jnp.int32)], vals)
            return carry

        lax.fori_loop(0, _CHUNK // _L, group_body, 0)

    def start(t):
        return pltpu.async_copy(
            table_hbm.at[sup_v.at[t]], bufs[t % 2], sems[t % 2])

    copies = {0: start(0)}
    for t in range(_N_CHUNKS):
        if t + 1 < _N_CHUNKS:
            copies[t + 1] = start(t + 1)
        copies[t].wait()
        select_chunk(t, bufs[t % 2])

    pltpu.sync_copy(out_v, out_hbm.at[pl.ds(wid * _B_PER_W, _B_PER_W)])


@jax.jit
def kernel(ixs, z):
    idx2d = ixs.astype(jnp.int32).reshape(BATCH // _CHUNK, _CHUNK)
    table2 = _pack(z.T)
    mesh = plsc.VectorSubcoreMesh(core_axis_name="c", subcore_axis_name="s")
    run = pl.kernel(
        _gather_body,
        out_type=jax.ShapeDtypeStruct((BATCH, DIM), jnp.float32),
        mesh=mesh,
        scratch_types=[
            pltpu.VMEM((_N_CHUNKS, _CHUNK), jnp.int32),
            pltpu.VMEM((_N_CHUNKS, _CHUNK), jnp.int32),
            pltpu.VMEM((_B_PER_W,), jnp.int32),
            pltpu.VMEM((_CHUNK, SUP), jnp.float32),
            pltpu.VMEM((_CHUNK, SUP), jnp.float32),
            pltpu.VMEM((_B_PER_W, DIM), jnp.float32),
            pltpu.SemaphoreType.DMA,
            pltpu.SemaphoreType.DMA,
        ],
        compiler_params=pltpu.CompilerParams(needs_layout_passes=False, disable_bounds_checks=True, disable_semaphore_checks=True),
    )
    return run(idx2d, table2)


# speedup vs baseline: 2.4981x; 1.0067x over previous
import jax
import jax.numpy as jnp
from jax import lax
from jax.experimental import pallas as pl
from jax.experimental.pallas import tpu as pltpu
from jax.experimental.pallas import tpu_sc as plsc

N_ROWS = 1_000_000
DIM = 32
BATCH = 16384

SEG = 262144
SEG_SHIFT = 18
SEG_MASK = SEG - 1
SUP = 128

_NC = 2
_NS = 16
_NW = _NC * _NS
_CHUNK = 128
_B_PER_W = BATCH // _NW
_N_CHUNKS = _B_PER_W // _CHUNK
_L = 16


_TBLK = 16384
_TGRID = SEG // _TBLK
_ZCB = N_ROWS // _TBLK
_TAIL_I = (N_ROWS - 3 * SEG) // _TBLK


def _pack_body(in0, in1, in2, in3, o_ref):
    i = pl.program_id(0)
    o_ref[:, 0:32] = in0[...].T
    o_ref[:, 32:64] = in1[...].T
    o_ref[:, 64:96] = in2[...].T

    @pl.when(i < _TAIL_I)
    def _():
        o_ref[:, 96:128] = in3[...].T

    @pl.when(i == _TAIL_I)
    def _():
        o_ref[0:576, 96:128] = in3[:, 0:576].T


def _pack(zt):
    return pl.pallas_call(
        _pack_body,
        grid=(_TGRID,),
        in_specs=[
            pl.BlockSpec((DIM, _TBLK), lambda i: (0, i)),
            pl.BlockSpec((DIM, _TBLK), lambda i: (0, i + _TGRID)),
            pl.BlockSpec((DIM, _TBLK), lambda i: (0, i + 2 * _TGRID)),
            pl.BlockSpec((DIM, _TBLK),
                         lambda i: (0, jnp.minimum(i + 3 * _TGRID, _ZCB))),
        ],
        out_specs=pl.BlockSpec((_TBLK, SUP), lambda i: (i, 0)),
        out_shape=jax.ShapeDtypeStruct((SEG, SUP), jnp.float32),
    )(zt, zt, zt, zt)


def _gather_body(idx_hbm, table_hbm, out_hbm, idx_v, sup_v, rem_v,
                 big0, big1, out_v, sem0, sem1):
    wid = lax.axis_index("s") * _NC + lax.axis_index("c")
    pltpu.sync_copy(idx_hbm.at[pl.ds(wid * _N_CHUNKS, _N_CHUNKS)], idx_v)

    for t in range(_N_CHUNKS):
        for k in range(_CHUNK // _L):
            v = idx_v[t, pl.ds(k * _L, _L)]
            sup_v[t, pl.ds(k * _L, _L)] = v & SEG_MASK
            rem_v[pl.ds((t * (_CHUNK // _L) + k) * _L, _L)] = v >> SEG_SHIFT

    bufs = (big0, big1)
    sems = (sem0, sem1)
    iota = lax.iota(jnp.int32, _L)

    def select_chunk(t, buf):
        def group_body(g, carry):
            lrow = g * _L + iota
            orow = t * _CHUNK + lrow
            rem16 = rem_v[pl.ds(t * _CHUNK + g * _L, _L)]
            col_base = rem16 * DIM
            for c in range(DIM):
                vals = plsc.load_gather(buf, [lrow, col_base + c])
                plsc.store_scatter(
                    out_v, [orow, jnp.full((_L,), c, jnp.int32)], vals)
            return carry

        lax.fori_loop(0, _CHUNK // _L, group_body, 0)

    def start(t):
        return pltpu.async_copy(
            table_hbm.at[sup_v.at[t]], bufs[t % 2], sems[t % 2])

    copies = {0: start(0)}
    for t in range(_N_CHUNKS):
        if t + 1 < _N_CHUNKS:
            copies[t + 1] = start(t + 1)
        copies[t].wait()
        select_chunk(t, bufs[t % 2])

    pltpu.sync_copy(out_v, out_hbm.at[pl.ds(wid * _B_PER_W, _B_PER_W)])


@jax.jit
def kernel(ixs, z):
    idx2d = ixs.astype(jnp.int32).reshape(BATCH // _CHUNK, _CHUNK)
    table2 = _pack(z.T)
    mesh = plsc.VectorSubcoreMesh(core_axis_name="c", subcore_axis_name="s")
    run = pl.kernel(
        _gather_body,
        out_type=jax.ShapeDtypeStruct((BATCH, DIM), jnp.float32),
        mesh=mesh,
        scratch_types=[
            pltpu.VMEM((_N_CHUNKS, _CHUNK), jnp.int32),
            pltpu.VMEM((_N_CHUNKS, _CHUNK), jnp.int32),
            pltpu.VMEM((_B_PER_W,), jnp.int32),
            pltpu.VMEM((_CHUNK, SUP), jnp.float32),
            pltpu.VMEM((_CHUNK, SUP), jnp.float32),
            pltpu.VMEM((_B_PER_W, DIM), jnp.float32),
            pltpu.SemaphoreType.DMA,
            pltpu.SemaphoreType.DMA,
        ],
        compiler_params=pltpu.CompilerParams(needs_layout_passes=False, disable_bounds_checks=True, disable_semaphore_checks=True),
    )
    return run(idx2d, table2)
